# Initial kernel scaffold; baseline (speedup 1.0000x reference)
import jax
import jax.numpy as jnp
from jax.experimental import pallas as pl


def kernel(x, edge_index, W1, b1, g1, be1, W2, b2, g2, be2, W3, b3, g3, be3, Wo, bo):
    raise NotImplementedError("write your pallas kernel here")


# speedup vs baseline: 17.7988x; 17.7988x over previous
import functools

import jax
import jax.numpy as jnp
from jax import lax
from jax.experimental import pallas as pl
from jax.experimental.pallas import tpu as pltpu
from jax.experimental.pallas import tpu_sc as plsc

_N = 10000
_E = 320000
_D = 128
_NC = 2
_NS = 16
_NW = _NC * _NS
_CH = 80
_NCH = _E // (_NW * _CH)
_SCC = 25
_NSC = _NCH // _SCC
_RPT = 624
_RPT_LAST = _N - _RPT * (_NS - 1)

_R = 1000
_G = _N // _R


def _init_rows(zeros_hbm, acc_sh, sid):
    base = sid * _RPT

    @pl.when(sid == _NS - 1)
    def _():
        pltpu.sync_copy(zeros_hbm, acc_sh.at[pl.ds(base, _RPT_LAST)])

    @pl.when(sid < _NS - 1)
    def _():
        pltpu.sync_copy(zeros_hbm.at[pl.ds(0, _RPT)],
                        acc_sh.at[pl.ds(base, _RPT)])


def _dump_rows(acc_sh, out_hbm, cid, sid):
    base = sid * _RPT

    @pl.when(sid == _NS - 1)
    def _():
        pltpu.sync_copy(acc_sh.at[pl.ds(base, _RPT_LAST)],
                        out_hbm.at[cid, pl.ds(base, _RPT_LAST)])

    @pl.when(sid < _NS - 1)
    def _():
        pltpu.sync_copy(acc_sh.at[pl.ds(base, _RPT)],
                        out_hbm.at[cid, pl.ds(base, _RPT)])


def _make_deg():
    mesh = plsc.VectorSubcoreMesh(core_axis_name="c", subcore_axis_name="s", num_cores=_NC, num_subcores=_NS)

    @functools.partial(
        pl.kernel,
        out_type=jax.ShapeDtypeStruct((_NC, _N, _D), jnp.float32),
        mesh=mesh,
        scratch_types=[
            pltpu.VMEM((_NCH, _CH), jnp.int32),
            pltpu.VMEM((_CH, _D), jnp.float32),
            pltpu.VMEM_SHARED((_N, _D), jnp.float32),
        ],
    )
    def deg_kernel(dst_hbm, ones_hbm, zeros_hbm, out_hbm, idx_v, ones_v, acc_sh):
        cid = lax.axis_index("c")
        sid = lax.axis_index("s")
        wid = sid * _NC + cid
        _init_rows(zeros_hbm, acc_sh, sid)
        pltpu.sync_copy(ones_hbm, ones_v)
        pltpu.sync_copy(dst_hbm.at[wid], idx_v)
        plsc.subcore_barrier()

        def body(c, carry):
            pltpu.sync_copy(ones_v, acc_sh.at[idx_v.at[c]], add=True)
            return carry

        lax.fori_loop(0, _NCH, body, 0)
        plsc.subcore_barrier()
        plsc.subcore_barrier()
        _dump_rows(acc_sh, out_hbm, cid, sid)

    return deg_kernel


def _make_agg():
    mesh = plsc.VectorSubcoreMesh(core_axis_name="c", subcore_axis_name="s", num_cores=_NC, num_subcores=_NS)

    @functools.partial(
        pl.kernel,
        out_type=jax.ShapeDtypeStruct((_NC, _N, _D), jnp.float32),
        mesh=mesh,
        scratch_types=[
            pltpu.VMEM((_SCC, _CH), jnp.int32),
            pltpu.VMEM((_SCC, _CH), jnp.int32),
            pltpu.VMEM((_CH, _D), jnp.float32),
            pltpu.VMEM((_CH, _D), jnp.float32),
            pltpu.SemaphoreType.DMA,
            pltpu.SemaphoreType.DMA,
            pltpu.VMEM_SHARED((_N, _D), jnp.float32),
        ],
    )
    def agg_kernel(hp_hbm, src_hbm, dst_hbm, zeros_hbm, out_hbm,
                   srcs, dsts, buf0, buf1, sem0, sem1, acc_sh):
        cid = lax.axis_index("c")
        sid = lax.axis_index("s")
        wid = sid * _NC + cid
        _init_rows(zeros_hbm, acc_sh, sid)
        plsc.subcore_barrier()

        bufs = (buf0, buf1)
        sems = (sem0, sem1)

        def superchunk(sc, carry):
            pltpu.sync_copy(src_hbm.at[wid, sc], srcs)
            pltpu.sync_copy(dst_hbm.at[wid, sc], dsts)
            pltpu.async_copy(hp_hbm.at[srcs.at[0]], buf0, sem0)
            pltpu.async_copy(hp_hbm.at[srcs.at[1]], buf1, sem1)

            def body(i, carry2):
                c = i * 2
                for b in range(2):
                    cc = c + b
                    pltpu.make_async_copy(hp_hbm.at[srcs.at[cc]],
                                          bufs[b], sems[b]).wait()
                    pltpu.sync_copy(bufs[b], acc_sh.at[dsts.at[cc]], add=True)
                    pltpu.async_copy(hp_hbm.at[srcs.at[cc + 2]], bufs[b], sems[b])
                return carry2

            loop = (_SCC - 3) // 2
            lax.fori_loop(0, loop, body, 0)
            t = loop * 2
            pltpu.make_async_copy(hp_hbm.at[srcs.at[t]], buf0, sem0).wait()
            pltpu.sync_copy(buf0, acc_sh.at[dsts.at[t]], add=True)
            pltpu.async_copy(hp_hbm.at[srcs.at[_SCC - 1]], buf0, sem0)
            pltpu.make_async_copy(hp_hbm.at[srcs.at[t + 1]], buf1, sem1).wait()
            pltpu.sync_copy(buf1, acc_sh.at[dsts.at[t + 1]], add=True)
            pltpu.make_async_copy(hp_hbm.at[srcs.at[_SCC - 1]], buf0, sem0).wait()
            pltpu.sync_copy(buf0, acc_sh.at[dsts.at[_SCC - 1]], add=True)
            return carry

        lax.fori_loop(0, _NSC, superchunk, 0)

        plsc.subcore_barrier()
        plsc.subcore_barrier()
        _dump_rows(acc_sh, out_hbm, cid, sid)

    return agg_kernel


_deg = _make_deg()
_agg = _make_agg()


def _k1_body(deg_ref, x_ref, w_ref, dis_ref, hp_ref):
    d = deg_ref[...]
    deg = d[0][:, :1] + d[1][:, :1]
    pos = deg > 0.0
    dis = jnp.where(pos, lax.rsqrt(jnp.where(pos, deg, 1.0)), 0.0)
    dis_b = jnp.broadcast_to(dis, (_R, _D))
    dis_ref[...] = dis_b
    hp_ref[...] = dis_b * jnp.dot(x_ref[...], w_ref[...],
                                  preferred_element_type=jnp.float32)


_k1 = pl.pallas_call(
    _k1_body,
    grid=(_G,),
    in_specs=[
        pl.BlockSpec((2, _R, _D), lambda i: (0, i, 0)),
        pl.BlockSpec((_R, _D), lambda i: (i, 0)),
        pl.BlockSpec((_D, _D), lambda i: (0, 0)),
    ],
    out_specs=[
        pl.BlockSpec((_R, _D), lambda i: (i, 0)),
        pl.BlockSpec((_R, _D), lambda i: (i, 0)),
    ],
    out_shape=[
        jax.ShapeDtypeStruct((_N, _D), jnp.float32),
        jax.ShapeDtypeStruct((_N, _D), jnp.float32),
    ],
)


def _epilogue(p, dis, b, g, be):
    h = dis * (p[0] + p[1]) + b
    mu = jnp.mean(h, axis=-1, keepdims=True)
    xc = h - mu
    var = jnp.mean(xc * xc, axis=-1, keepdims=True)
    h = xc * lax.rsqrt(var + 1e-5) * g + be
    return jnp.maximum(h, 0.0)


def _mid_body(p_ref, dis_ref, b_ref, g_ref, be_ref, w_ref, out_ref):
    dis = dis_ref[...]
    h = _epilogue(p_ref[...], dis, b_ref[...], g_ref[...], be_ref[...])
    out_ref[...] = dis * jnp.dot(h, w_ref[...], preferred_element_type=jnp.float32)


_mid = pl.pallas_call(
    _mid_body,
    grid=(_G,),
    in_specs=[
        pl.BlockSpec((2, _R, _D), lambda i: (0, i, 0)),
        pl.BlockSpec((_R, _D), lambda i: (i, 0)),
        pl.BlockSpec((1, _D), lambda i: (0, 0)),
        pl.BlockSpec((1, _D), lambda i: (0, 0)),
        pl.BlockSpec((1, _D), lambda i: (0, 0)),
        pl.BlockSpec((_D, _D), lambda i: (0, 0)),
    ],
    out_specs=pl.BlockSpec((_R, _D), lambda i: (i, 0)),
    out_shape=jax.ShapeDtypeStruct((_N, _D), jnp.float32),
)

_NCLS = 40


def _fin_body(p_ref, dis_ref, b_ref, g_ref, be_ref, wo_ref, bo_ref, out_ref):
    h = _epilogue(p_ref[...], dis_ref[...], b_ref[...], g_ref[...], be_ref[...])
    logits = jnp.dot(h, wo_ref[...], preferred_element_type=jnp.float32) + bo_ref[...]
    m = jnp.max(logits, axis=-1, keepdims=True)
    s = logits - m
    lse = jnp.log(jnp.sum(jnp.exp(s), axis=-1, keepdims=True))
    out_ref[...] = s - lse


_fin = pl.pallas_call(
    _fin_body,
    grid=(_G,),
    in_specs=[
        pl.BlockSpec((2, _R, _D), lambda i: (0, i, 0)),
        pl.BlockSpec((_R, _D), lambda i: (i, 0)),
        pl.BlockSpec((1, _D), lambda i: (0, 0)),
        pl.BlockSpec((1, _D), lambda i: (0, 0)),
        pl.BlockSpec((1, _D), lambda i: (0, 0)),
        pl.BlockSpec((_D, _NCLS), lambda i: (0, 0)),
        pl.BlockSpec((1, _NCLS), lambda i: (0, 0)),
    ],
    out_specs=pl.BlockSpec((_R, _NCLS), lambda i: (i, 0)),
    out_shape=jax.ShapeDtypeStruct((_N, _NCLS), jnp.float32),
)


def kernel(x, edge_index, W1, b1, g1, be1, W2, b2, g2, be2, W3, b3, g3, be3, Wo, bo):
    src = edge_index[0].reshape(_NW, _NSC, _SCC, _CH)
    dst = edge_index[1].reshape(_NW, _NSC, _SCC, _CH)
    dst3 = edge_index[1].reshape(_NW, _NCH, _CH)
    zerosD = jnp.zeros((_RPT_LAST, _D), jnp.float32)
    onesD = jnp.ones((_CH, _D), jnp.float32)

    deg_p = _deg(dst3, onesD, zerosD)
    dis_b, hp = _k1(deg_p, x, W1)
    for (W, b, g, be) in ((W2, b1, g1, be1), (W3, b2, g2, be2)):
        p = _agg(hp, src, dst, zerosD)
        hp = _mid(p, dis_b, b.reshape(1, _D), g.reshape(1, _D),
                  be.reshape(1, _D), W)
    p = _agg(hp, src, dst, zerosD)
    return _fin(p, dis_b, b3.reshape(1, _D), g3.reshape(1, _D),
                be3.reshape(1, _D), Wo, bo.reshape(1, _NCLS))


# speedup vs baseline: 19.8665x; 1.1162x over previous
import functools

import jax
import jax.numpy as jnp
from jax import lax
from jax.experimental import pallas as pl
from jax.experimental.pallas import tpu as pltpu
from jax.experimental.pallas import tpu_sc as plsc

_N = 10000
_E = 320000
_D = 128
_NC = 2
_NS = 16
_NW = _NC * _NS
_CH = 80
_NCH = _E // (_NW * _CH)
_SCC = 25
_NSC = _NCH // _SCC
_RPT = 624
_RPT_LAST = _N - _RPT * (_NS - 1)

_R = 1000
_G = _N // _R


def _init_rows(zeros_hbm, acc_sh, sid):
    base = sid * _RPT

    @pl.when(sid == _NS - 1)
    def _():
        pltpu.sync_copy(zeros_hbm, acc_sh.at[pl.ds(base, _RPT_LAST)])

    @pl.when(sid < _NS - 1)
    def _():
        pltpu.sync_copy(zeros_hbm.at[pl.ds(0, _RPT)],
                        acc_sh.at[pl.ds(base, _RPT)])


def _dump_rows(acc_sh, out_hbm, cid, sid):
    base = sid * _RPT

    @pl.when(sid == _NS - 1)
    def _():
        pltpu.sync_copy(acc_sh.at[pl.ds(base, _RPT_LAST)],
                        out_hbm.at[cid, pl.ds(base, _RPT_LAST)])

    @pl.when(sid < _NS - 1)
    def _():
        pltpu.sync_copy(acc_sh.at[pl.ds(base, _RPT)],
                        out_hbm.at[cid, pl.ds(base, _RPT)])


def _make_deg():
    mesh = plsc.VectorSubcoreMesh(core_axis_name="c", subcore_axis_name="s", num_cores=_NC, num_subcores=_NS)

    @functools.partial(
        pl.kernel,
        out_type=jax.ShapeDtypeStruct((_NC, _N, _D), jnp.float32),
        mesh=mesh,
        scratch_types=[
            pltpu.VMEM((_NCH, _CH), jnp.int32),
            pltpu.VMEM((_CH, _D), jnp.float32),
            pltpu.SemaphoreType.DMA,
            pltpu.VMEM_SHARED((_N, _D), jnp.float32),
        ],
    )
    def deg_kernel(dst_hbm, ones_hbm, zeros_hbm, out_hbm, idx_v, ones_v, sem, acc_sh):
        cid = lax.axis_index("c")
        sid = lax.axis_index("s")
        wid = sid * _NC + cid
        _init_rows(zeros_hbm, acc_sh, sid)
        pltpu.sync_copy(ones_hbm, ones_v)
        pltpu.sync_copy(dst_hbm.at[wid], idx_v)
        plsc.subcore_barrier()

        def body(g, carry):
            c = g * 5
            for j in range(5):
                pltpu.async_copy(ones_v, acc_sh.at[idx_v.at[c + j]], sem,
                                 add=True)
            for j in range(5):
                pltpu.make_async_copy(ones_v, acc_sh.at[idx_v.at[c + j]],
                                      sem).wait()
            return carry

        lax.fori_loop(0, _NCH // 5, body, 0)
        plsc.subcore_barrier()
        plsc.subcore_barrier()
        _dump_rows(acc_sh, out_hbm, cid, sid)

    return deg_kernel


def _make_agg():
    mesh = plsc.VectorSubcoreMesh(core_axis_name="c", subcore_axis_name="s", num_cores=_NC, num_subcores=_NS)

    @functools.partial(
        pl.kernel,
        out_type=jax.ShapeDtypeStruct((_NC, _N, _D), jnp.float32),
        mesh=mesh,
        scratch_types=[
            pltpu.VMEM((_SCC, _CH), jnp.int32),
            pltpu.VMEM((_SCC, _CH), jnp.int32),
            pltpu.VMEM((_CH, _D), jnp.float32),
            pltpu.VMEM((_CH, _D), jnp.float32),
            pltpu.VMEM((_CH, _D), jnp.float32),
            pltpu.SemaphoreType.DMA,
            pltpu.SemaphoreType.DMA,
            pltpu.SemaphoreType.DMA,
            pltpu.VMEM_SHARED((_N, _D), jnp.float32),
        ],
    )
    def agg_kernel(hp_hbm, src_hbm, dst_hbm, zeros_hbm, out_hbm,
                   srcs, dsts, buf0, buf1, buf2, sem0, sem1, sem2, acc_sh):
        cid = lax.axis_index("c")
        sid = lax.axis_index("s")
        wid = sid * _NC + cid
        _init_rows(zeros_hbm, acc_sh, sid)
        plsc.subcore_barrier()

        bufs = (buf0, buf1, buf2)
        sems = (sem0, sem1, sem2)

        def superchunk(sc, carry):
            pltpu.sync_copy(src_hbm.at[wid, sc], srcs)
            pltpu.sync_copy(dst_hbm.at[wid, sc], dsts)
            for b in range(3):
                pltpu.async_copy(hp_hbm.at[srcs.at[b]], bufs[b], sems[b])

            def body(i, carry2):
                c = i * 3
                for b in range(3):
                    cc = c + b
                    pltpu.make_async_copy(hp_hbm.at[srcs.at[cc]],
                                          bufs[b], sems[b]).wait()
                    pltpu.sync_copy(bufs[b], acc_sh.at[dsts.at[cc]], add=True)
                    pltpu.async_copy(hp_hbm.at[srcs.at[cc + 3]], bufs[b], sems[b])
                return carry2

            loop = (_SCC - 4) // 3
            lax.fori_loop(0, loop, body, 0)
            t = loop * 3
            pltpu.make_async_copy(hp_hbm.at[srcs.at[t]], buf0, sem0).wait()
            pltpu.sync_copy(buf0, acc_sh.at[dsts.at[t]], add=True)
            pltpu.async_copy(hp_hbm.at[srcs.at[_SCC - 1]], buf0, sem0)
            pltpu.make_async_copy(hp_hbm.at[srcs.at[t + 1]], buf1, sem1).wait()
            pltpu.sync_copy(buf1, acc_sh.at[dsts.at[t + 1]], add=True)
            pltpu.make_async_copy(hp_hbm.at[srcs.at[t + 2]], buf2, sem2).wait()
            pltpu.sync_copy(buf2, acc_sh.at[dsts.at[t + 2]], add=True)
            pltpu.make_async_copy(hp_hbm.at[srcs.at[_SCC - 1]], buf0, sem0).wait()
            pltpu.sync_copy(buf0, acc_sh.at[dsts.at[_SCC - 1]], add=True)
            return carry

        lax.fori_loop(0, _NSC, superchunk, 0)

        plsc.subcore_barrier()
        plsc.subcore_barrier()
        _dump_rows(acc_sh, out_hbm, cid, sid)

    return agg_kernel


_deg = _make_deg()
_agg = _make_agg()


def _k1_body(deg_ref, x_ref, w_ref, dis_ref, hp_ref):
    d = deg_ref[...]
    deg = d[0][:, :1] + d[1][:, :1]
    pos = deg > 0.0
    dis = jnp.where(pos, lax.rsqrt(jnp.where(pos, deg, 1.0)), 0.0)
    dis_b = jnp.broadcast_to(dis, (_R, _D))
    dis_ref[...] = dis_b
    hp_ref[...] = dis_b * jnp.dot(x_ref[...], w_ref[...],
                                  preferred_element_type=jnp.float32)


_k1 = pl.pallas_call(
    _k1_body,
    grid=(_G,),
    in_specs=[
        pl.BlockSpec((2, _R, _D), lambda i: (0, i, 0)),
        pl.BlockSpec((_R, _D), lambda i: (i, 0)),
        pl.BlockSpec((_D, _D), lambda i: (0, 0)),
    ],
    out_specs=[
        pl.BlockSpec((_R, _D), lambda i: (i, 0)),
        pl.BlockSpec((_R, _D), lambda i: (i, 0)),
    ],
    out_shape=[
        jax.ShapeDtypeStruct((_N, _D), jnp.float32),
        jax.ShapeDtypeStruct((_N, _D), jnp.float32),
    ],
)


def _epilogue(p, dis, b, g, be):
    h = dis * (p[0] + p[1]) + b
    mu = jnp.mean(h, axis=-1, keepdims=True)
    xc = h - mu
    var = jnp.mean(xc * xc, axis=-1, keepdims=True)
    h = xc * lax.rsqrt(var + 1e-5) * g + be
    return jnp.maximum(h, 0.0)


def _mid_body(p_ref, dis_ref, b_ref, g_ref, be_ref, w_ref, out_ref):
    dis = dis_ref[...]
    h = _epilogue(p_ref[...], dis, b_ref[...], g_ref[...], be_ref[...])
    out_ref[...] = dis * jnp.dot(h, w_ref[...], preferred_element_type=jnp.float32)


_mid = pl.pallas_call(
    _mid_body,
    grid=(_G,),
    in_specs=[
        pl.BlockSpec((2, _R, _D), lambda i: (0, i, 0)),
        pl.BlockSpec((_R, _D), lambda i: (i, 0)),
        pl.BlockSpec((1, _D), lambda i: (0, 0)),
        pl.BlockSpec((1, _D), lambda i: (0, 0)),
        pl.BlockSpec((1, _D), lambda i: (0, 0)),
        pl.BlockSpec((_D, _D), lambda i: (0, 0)),
    ],
    out_specs=pl.BlockSpec((_R, _D), lambda i: (i, 0)),
    out_shape=jax.ShapeDtypeStruct((_N, _D), jnp.float32),
)

_NCLS = 40


def _fin_body(p_ref, dis_ref, b_ref, g_ref, be_ref, wo_ref, bo_ref, out_ref):
    h = _epilogue(p_ref[...], dis_ref[...], b_ref[...], g_ref[...], be_ref[...])
    logits = jnp.dot(h, wo_ref[...], preferred_element_type=jnp.float32) + bo_ref[...]
    m = jnp.max(logits, axis=-1, keepdims=True)
    s = logits - m
    lse = jnp.log(jnp.sum(jnp.exp(s), axis=-1, keepdims=True))
    out_ref[...] = s - lse


_fin = pl.pallas_call(
    _fin_body,
    grid=(_G,),
    in_specs=[
        pl.BlockSpec((2, _R, _D), lambda i: (0, i, 0)),
        pl.BlockSpec((_R, _D), lambda i: (i, 0)),
        pl.BlockSpec((1, _D), lambda i: (0, 0)),
        pl.BlockSpec((1, _D), lambda i: (0, 0)),
        pl.BlockSpec((1, _D), lambda i: (0, 0)),
        pl.BlockSpec((_D, _NCLS), lambda i: (0, 0)),
        pl.BlockSpec((1, _NCLS), lambda i: (0, 0)),
    ],
    out_specs=pl.BlockSpec((_R, _NCLS), lambda i: (i, 0)),
    out_shape=jax.ShapeDtypeStruct((_N, _NCLS), jnp.float32),
)


def kernel(x, edge_index, W1, b1, g1, be1, W2, b2, g2, be2, W3, b3, g3, be3, Wo, bo):
    src = edge_index[0].reshape(_NW, _NSC, _SCC, _CH)
    dst = edge_index[1].reshape(_NW, _NSC, _SCC, _CH)
    dst3 = edge_index[1].reshape(_NW, _NCH, _CH)
    zerosD = jnp.zeros((_RPT_LAST, _D), jnp.float32)
    onesD = jnp.ones((_CH, _D), jnp.float32)

    deg_p = _deg(dst3, onesD, zerosD)
    dis_b, hp = _k1(deg_p, x, W1)
    for (W, b, g, be) in ((W2, b1, g1, be1), (W3, b2, g2, be2)):
        p = _agg(hp, src, dst, zerosD)
        hp = _mid(p, dis_b, b.reshape(1, _D), g.reshape(1, _D),
                  be.reshape(1, _D), W)
    p = _agg(hp, src, dst, zerosD)
    return _fin(p, dis_b, b3.reshape(1, _D), g3.reshape(1, _D),
                be3.reshape(1, _D), Wo, bo.reshape(1, _NCLS))


# speedup vs baseline: 21.8728x; 1.1010x over previous
import functools

import jax
import jax.numpy as jnp
from jax import lax
from jax.experimental import pallas as pl
from jax.experimental.pallas import tpu as pltpu
from jax.experimental.pallas import tpu_sc as plsc

_N = 10000
_E = 320000
_D = 128
_NC = 2
_NS = 16
_NW = _NC * _NS
_CH = 80
_NCH = _E // (_NW * _CH)
_SCC = 25
_NSC = _NCH // _SCC
_RPT = 624
_RPT_LAST = _N - _RPT * (_NS - 1)

_R = 1000
_G = _N // _R


def _init_rows(zeros_hbm, acc_sh, sid):
    base = sid * _RPT

    @pl.when(sid == _NS - 1)
    def _():
        pltpu.sync_copy(zeros_hbm, acc_sh.at[pl.ds(base, _RPT_LAST)])

    @pl.when(sid < _NS - 1)
    def _():
        pltpu.sync_copy(zeros_hbm.at[pl.ds(0, _RPT)],
                        acc_sh.at[pl.ds(base, _RPT)])


def _dump_rows(acc_sh, out_hbm, cid, sid):
    base = sid * _RPT

    @pl.when(sid == _NS - 1)
    def _():
        pltpu.sync_copy(acc_sh.at[pl.ds(base, _RPT_LAST)],
                        out_hbm.at[cid, pl.ds(base, _RPT_LAST)])

    @pl.when(sid < _NS - 1)
    def _():
        pltpu.sync_copy(acc_sh.at[pl.ds(base, _RPT)],
                        out_hbm.at[cid, pl.ds(base, _RPT)])


def _make_degreg():
    mesh = plsc.VectorSubcoreMesh(core_axis_name="c", subcore_axis_name="s",
                                  num_cores=_NC, num_subcores=_NS)

    @functools.partial(
        pl.kernel,
        out_type=jax.ShapeDtypeStruct((_NW * 10240,), jnp.float32),
        mesh=mesh,
        compiler_params=pltpu.CompilerParams(needs_layout_passes=False),
        scratch_types=[
            pltpu.VMEM((_NCH, _CH), jnp.int32),
            pltpu.VMEM((10240,), jnp.float32),
        ],
    )
    def degreg_kernel(dst_hbm, zeros_hbm, out_hbm, idx_v, acc_v):
        cid = lax.axis_index("c")
        sid = lax.axis_index("s")
        wid = sid * _NC + cid
        pltpu.sync_copy(zeros_hbm, acc_v)
        pltpu.sync_copy(dst_hbm.at[wid], idx_v)
        ones = jnp.ones((16,), jnp.float32)

        def body(k, carry):
            for j in range(5):
                v = idx_v[k, pl.ds(j * 16, 16)]
                plsc.addupdate_scatter(acc_v, [v], ones)
            return carry

        lax.fori_loop(0, _NCH, body, 0)
        pltpu.sync_copy(acc_v, out_hbm.at[pl.ds(wid * 10240, 10240)])

    return degreg_kernel


def _make_agg():
    mesh = plsc.VectorSubcoreMesh(core_axis_name="c", subcore_axis_name="s", num_cores=_NC, num_subcores=_NS)

    @functools.partial(
        pl.kernel,
        out_type=jax.ShapeDtypeStruct((_NC, _N, _D), jnp.float32),
        mesh=mesh,
        scratch_types=[
            pltpu.VMEM((_SCC, _CH), jnp.int32),
            pltpu.VMEM((_SCC, _CH), jnp.int32),
            pltpu.VMEM((_CH, _D), jnp.float32),
            pltpu.VMEM((_CH, _D), jnp.float32),
            pltpu.VMEM((_CH, _D), jnp.float32),
            pltpu.SemaphoreType.DMA,
            pltpu.SemaphoreType.DMA,
            pltpu.SemaphoreType.DMA,
            pltpu.VMEM_SHARED((_N, _D), jnp.float32),
        ],
    )
    def agg_kernel(hp_hbm, src_hbm, dst_hbm, zeros_hbm, out_hbm,
                   srcs, dsts, buf0, buf1, buf2, sem0, sem1, sem2, acc_sh):
        cid = lax.axis_index("c")
        sid = lax.axis_index("s")
        wid = sid * _NC + cid
        _init_rows(zeros_hbm, acc_sh, sid)
        plsc.subcore_barrier()

        bufs = (buf0, buf1, buf2)
        sems = (sem0, sem1, sem2)

        def superchunk(sc, carry):
            pltpu.sync_copy(src_hbm.at[wid, sc], srcs)
            pltpu.sync_copy(dst_hbm.at[wid, sc], dsts)
            for b in range(3):
                pltpu.async_copy(hp_hbm.at[srcs.at[b]], bufs[b], sems[b])

            def body(i, carry2):
                c = i * 3
                for b in range(3):
                    cc = c + b
                    pltpu.make_async_copy(hp_hbm.at[srcs.at[cc]],
                                          bufs[b], sems[b]).wait()
                    pltpu.sync_copy(bufs[b], acc_sh.at[dsts.at[cc]], add=True)
                    pltpu.async_copy(hp_hbm.at[srcs.at[cc + 3]], bufs[b], sems[b])
                return carry2

            loop = (_SCC - 4) // 3
            lax.fori_loop(0, loop, body, 0)
            t = loop * 3
            pltpu.make_async_copy(hp_hbm.at[srcs.at[t]], buf0, sem0).wait()
            pltpu.sync_copy(buf0, acc_sh.at[dsts.at[t]], add=True)
            pltpu.async_copy(hp_hbm.at[srcs.at[_SCC - 1]], buf0, sem0)
            pltpu.make_async_copy(hp_hbm.at[srcs.at[t + 1]], buf1, sem1).wait()
            pltpu.sync_copy(buf1, acc_sh.at[dsts.at[t + 1]], add=True)
            pltpu.make_async_copy(hp_hbm.at[srcs.at[t + 2]], buf2, sem2).wait()
            pltpu.sync_copy(buf2, acc_sh.at[dsts.at[t + 2]], add=True)
            pltpu.make_async_copy(hp_hbm.at[srcs.at[_SCC - 1]], buf0, sem0).wait()
            pltpu.sync_copy(buf0, acc_sh.at[dsts.at[_SCC - 1]], add=True)
            return carry

        lax.fori_loop(0, _NSC, superchunk, 0)

        plsc.subcore_barrier()
        plsc.subcore_barrier()
        _dump_rows(acc_sh, out_hbm, cid, sid)

    return agg_kernel


_degreg = _make_degreg()
_agg = _make_agg()


def _dsum_body(dp_ref, dis_ref):
    deg = jnp.sum(dp_ref[...], axis=0, keepdims=True)
    pos = deg > 0.0
    dis_ref[...] = jnp.where(pos, lax.rsqrt(jnp.where(pos, deg, 1.0)), 0.0)


_dsum = pl.pallas_call(
    _dsum_body,
    grid=(8,),
    in_specs=[pl.BlockSpec((_NW, 1280), lambda i: (0, i))],
    out_specs=pl.BlockSpec((1, 1280), lambda i: (0, i)),
    out_shape=jax.ShapeDtypeStruct((1, 10240), jnp.float32),
)


def _k1_body(dis_ref, x_ref, w_ref, hp_ref):
    dis_b = jnp.broadcast_to(dis_ref[...], (_R, _D))
    hp_ref[...] = dis_b * jnp.dot(x_ref[...], w_ref[...],
                                  preferred_element_type=jnp.float32)


_k1 = pl.pallas_call(
    _k1_body,
    grid=(_G,),
    in_specs=[
        pl.BlockSpec((_R, 1), lambda i: (i, 0)),
        pl.BlockSpec((_R, _D), lambda i: (i, 0)),
        pl.BlockSpec((_D, _D), lambda i: (0, 0)),
    ],
    out_specs=pl.BlockSpec((_R, _D), lambda i: (i, 0)),
    out_shape=jax.ShapeDtypeStruct((_N, _D), jnp.float32),
)


def _epilogue(p, dis, b, g, be):
    h = dis * (p[0] + p[1]) + b
    mu = jnp.mean(h, axis=-1, keepdims=True)
    xc = h - mu
    var = jnp.mean(xc * xc, axis=-1, keepdims=True)
    h = xc * lax.rsqrt(var + 1e-5) * g + be
    return jnp.maximum(h, 0.0)


def _mid_body(p_ref, dis_ref, b_ref, g_ref, be_ref, w_ref, out_ref):
    dis = jnp.broadcast_to(dis_ref[...], (_R, _D))
    h = _epilogue(p_ref[...], dis, b_ref[...], g_ref[...], be_ref[...])
    out_ref[...] = dis * jnp.dot(h, w_ref[...], preferred_element_type=jnp.float32)


_mid = pl.pallas_call(
    _mid_body,
    grid=(_G,),
    in_specs=[
        pl.BlockSpec((2, _R, _D), lambda i: (0, i, 0)),
        pl.BlockSpec((_R, 1), lambda i: (i, 0)),
        pl.BlockSpec((1, _D), lambda i: (0, 0)),
        pl.BlockSpec((1, _D), lambda i: (0, 0)),
        pl.BlockSpec((1, _D), lambda i: (0, 0)),
        pl.BlockSpec((_D, _D), lambda i: (0, 0)),
    ],
    out_specs=pl.BlockSpec((_R, _D), lambda i: (i, 0)),
    out_shape=jax.ShapeDtypeStruct((_N, _D), jnp.float32),
)

_NCLS = 40


def _fin_body(p_ref, dis_ref, b_ref, g_ref, be_ref, wo_ref, bo_ref, out_ref):
    h = _epilogue(p_ref[...], jnp.broadcast_to(dis_ref[...], (_R, _D)),
                  b_ref[...], g_ref[...], be_ref[...])
    logits = jnp.dot(h, wo_ref[...], preferred_element_type=jnp.float32) + bo_ref[...]
    m = jnp.max(logits, axis=-1, keepdims=True)
    s = logits - m
    lse = jnp.log(jnp.sum(jnp.exp(s), axis=-1, keepdims=True))
    out_ref[...] = s - lse


_fin = pl.pallas_call(
    _fin_body,
    grid=(_G,),
    in_specs=[
        pl.BlockSpec((2, _R, _D), lambda i: (0, i, 0)),
        pl.BlockSpec((_R, 1), lambda i: (i, 0)),
        pl.BlockSpec((1, _D), lambda i: (0, 0)),
        pl.BlockSpec((1, _D), lambda i: (0, 0)),
        pl.BlockSpec((1, _D), lambda i: (0, 0)),
        pl.BlockSpec((_D, _NCLS), lambda i: (0, 0)),
        pl.BlockSpec((1, _NCLS), lambda i: (0, 0)),
    ],
    out_specs=pl.BlockSpec((_R, _NCLS), lambda i: (i, 0)),
    out_shape=jax.ShapeDtypeStruct((_N, _NCLS), jnp.float32),
)


def kernel(x, edge_index, W1, b1, g1, be1, W2, b2, g2, be2, W3, b3, g3, be3, Wo, bo):
    src = edge_index[0].reshape(_NW, _NSC, _SCC, _CH)
    dst = edge_index[1].reshape(_NW, _NSC, _SCC, _CH)
    dst3 = edge_index[1].reshape(_NW, _NCH, _CH)
    zerosD = jnp.zeros((_RPT_LAST, _D), jnp.float32)
    zeros1 = jnp.zeros((10240,), jnp.float32)

    deg_reg = _degreg(dst3, zeros1)
    dis_b = _dsum(deg_reg.reshape(_NW, 10240)).reshape(10240)[:_N, None]
    hp = _k1(dis_b, x, W1)
    for (W, b, g, be) in ((W2, b1, g1, be1), (W3, b2, g2, be2)):
        p = _agg(hp, src, dst, zerosD)
        hp = _mid(p, dis_b, b.reshape(1, _D), g.reshape(1, _D),
                  be.reshape(1, _D), W)
    p = _agg(hp, src, dst, zerosD)
    return _fin(p, dis_b, b3.reshape(1, _D), g3.reshape(1, _D),
                be3.reshape(1, _D), Wo, bo.reshape(1, _NCLS))
